# baseline (speedup 1.0000x reference)
import jax
import jax.numpy as jnp
from jax.experimental import pallas as pl
from jax.experimental.pallas import tpu as pltpu

CLASSES = 128
CH = 128
EPS = 1e-5
WT = 2048


def _make_kernel(n_tok):
    inv_n = 1.0 / float(n_tok)

    def body(x_ref, w1_ref, b1_ref, g_ref, be_ref, wall_ref, ball_ref,
             out_ref, mask_ref, sy_ref, syy_ref, sc_ref, sh_ref):
        p = pl.program_id(0)
        first = jnp.logical_and(pl.program_id(1) == 0, pl.program_id(2) == 0)

        y = jax.lax.dot_general(
            w1_ref[...], x_ref[0].astype(jnp.bfloat16), (((1,), (0,)), ((), ())),
            preferred_element_type=jnp.float32) + b1_ref[...]

        @pl.when(p == 0)
        def _stats():
            @pl.when(first)
            def _():
                sy_ref[...] = jnp.zeros_like(sy_ref)
                syy_ref[...] = jnp.zeros_like(syy_ref)

            sy_ref[...] += jnp.sum(y, axis=1, keepdims=True)
            syy_ref[...] += jnp.sum(y * y, axis=1, keepdims=True)

        @pl.when(p == 1)
        def _fused():
            @pl.when(first)
            def _():
                mu = sy_ref[...] * inv_n
                var = syy_ref[...] * inv_n - mu * mu
                scale = g_ref[...] * jax.lax.rsqrt(var + EPS)
                sc_ref[...] = scale
                sh_ref[...] = be_ref[...] - mu * scale

            yn = y * sc_ref[...] + sh_ref[...]
            x_lat = jnp.where(yn >= 0, yn, 0.01 * yn).astype(jnp.bfloat16)
            z = jax.lax.dot_general(
                wall_ref[...], x_lat, (((1,), (0,)), ((), ())),
                preferred_element_type=jnp.float32) + ball_ref[...]
            logits = z[0:CLASSES, :]
            m = jnp.max(logits, axis=0, keepdims=True)
            row = jax.lax.broadcasted_iota(jnp.int32, logits.shape, 0)
            ind = jnp.min(jnp.where(logits == m, row, CLASSES),
                          axis=0, keepdims=True)
            allreg = z[CLASSES + 8:CLASSES + 8 + CLASSES, :]
            reg = jnp.sum(jnp.where(row == ind, allreg, 0.0),
                          axis=0, keepdims=True)
            out_ref[0] = (ind.astype(jnp.float32) + reg) * (1.0 / float(CLASSES))
            mz = z[CLASSES:CLASSES + 1, :]
            mask_ref[0] = jnp.where(mz >= 0, mz, 0.01 * mz)

    return body


def kernel(x, conv1_w, conv1_b, bn_gamma, bn_beta, conv2_w, conv2_b, cm_w, cm_b):
    B, CIN, H, W = x.shape
    n_w = W // WT
    xr = x.reshape(B, CIN, W)
    n_tok = B * H * W

    w1b = conv1_w.astype(jnp.bfloat16)
    w_all = jnp.concatenate(
        [conv2_w, jnp.zeros((7, CH), jnp.float32), cm_w[:, :, 0]],
        axis=0).astype(jnp.bfloat16)
    b_all = jnp.concatenate(
        [conv2_b, jnp.zeros((7,), jnp.float32), cm_b[:, 0]], axis=0)[:, None]

    col = lambda a: a[:, None]
    out, mask = pl.pallas_call(
        _make_kernel(n_tok),
        grid=(2, B, n_w),
        in_specs=[
            pl.BlockSpec((1, CIN, WT), lambda p, b, w: (b, 0, w)),
            pl.BlockSpec((CH, CIN), lambda p, b, w: (0, 0)),
            pl.BlockSpec((CH, 1), lambda p, b, w: (0, 0)),
            pl.BlockSpec((CH, 1), lambda p, b, w: (0, 0)),
            pl.BlockSpec((CH, 1), lambda p, b, w: (0, 0)),
            pl.BlockSpec((2 * CLASSES + 8, CH), lambda p, b, w: (0, 0)),
            pl.BlockSpec((2 * CLASSES + 8, 1), lambda p, b, w: (0, 0)),
        ],
        out_specs=[
            pl.BlockSpec((1, 1, WT), lambda p, b, w: (b * n_w + w, 0, 0)),
            pl.BlockSpec((1, 1, WT), lambda p, b, w: (b * n_w + w, 0, 0)),
        ],
        out_shape=[
            jax.ShapeDtypeStruct((B * n_w, 1, WT), jnp.float32),
            jax.ShapeDtypeStruct((B * n_w, 1, WT), jnp.float32),
        ],
        scratch_shapes=[
            pltpu.VMEM((CH, 1), jnp.float32),
            pltpu.VMEM((CH, 1), jnp.float32),
            pltpu.VMEM((CH, 1), jnp.float32),
            pltpu.VMEM((CH, 1), jnp.float32),
        ],
        compiler_params=pltpu.CompilerParams(
            dimension_semantics=("arbitrary", "arbitrary", "arbitrary")),
    )(xr, w1b, col(conv1_b), col(bn_gamma), col(bn_beta), w_all, b_all)

    return out.reshape(B, 1, 1, W), mask.reshape(B, 1, 1, W)


# speedup vs baseline: 1.2621x; 1.2621x over previous
import jax
import jax.numpy as jnp
from jax.experimental import pallas as pl
from jax.experimental.pallas import tpu as pltpu

CLASSES = 128
CH = 128
EPS = 1e-5
WT = 2048


def _make_kernel(n_tok):
    inv_n = 1.0 / float(n_tok)

    def body(x_ref, w1_ref, b1_ref, g_ref, be_ref, wall_ref, ball_ref,
             out_ref, mask_ref, sy_ref, syy_ref, sc_ref, sh_ref):
        p = pl.program_id(0)
        first = jnp.logical_and(pl.program_id(1) == 0, pl.program_id(2) == 0)

        xt = x_ref[0].reshape(CH, WT).astype(jnp.bfloat16)
        y = jax.lax.dot_general(
            w1_ref[...], xt, (((1,), (0,)), ((), ())),
            preferred_element_type=jnp.float32) + b1_ref[...]

        @pl.when(p == 0)
        def _stats():
            @pl.when(first)
            def _():
                sy_ref[...] = jnp.zeros_like(sy_ref)
                syy_ref[...] = jnp.zeros_like(syy_ref)

            sy_ref[...] += jnp.sum(y, axis=1, keepdims=True)
            syy_ref[...] += jnp.sum(y * y, axis=1, keepdims=True)

        @pl.when(p == 1)
        def _fused():
            @pl.when(first)
            def _():
                mu = sy_ref[...] * inv_n
                var = syy_ref[...] * inv_n - mu * mu
                scale = g_ref[...] * jax.lax.rsqrt(var + EPS)
                sc_ref[...] = scale
                sh_ref[...] = be_ref[...] - mu * scale

            yn = y * sc_ref[...] + sh_ref[...]
            x_lat = jnp.where(yn >= 0, yn, 0.01 * yn).astype(jnp.bfloat16)
            z = jax.lax.dot_general(
                wall_ref[...], x_lat, (((1,), (0,)), ((), ())),
                preferred_element_type=jnp.float32) + ball_ref[...]
            logits = z[0:CLASSES, :]
            m = jnp.max(logits, axis=0, keepdims=True)
            row = jax.lax.broadcasted_iota(jnp.int32, logits.shape, 0)
            ind = jnp.min(jnp.where(logits == m, row, CLASSES),
                          axis=0, keepdims=True)
            allreg = z[CLASSES + 8:CLASSES + 8 + CLASSES, :]
            reg = jnp.sum(jnp.where(row == ind, allreg, 0.0),
                          axis=0, keepdims=True)
            out_ref[0] = (ind.astype(jnp.float32) + reg) * (1.0 / float(CLASSES))
            mz = z[CLASSES:CLASSES + 1, :]
            mask_ref[0] = jnp.where(mz >= 0, mz, 0.01 * mz)

    return body


def kernel(x, conv1_w, conv1_b, bn_gamma, bn_beta, conv2_w, conv2_b, cm_w, cm_b):
    B, CIN, H, W = x.shape
    n_w = W // WT
    xv = x.reshape(B, CIN, W // 128, 128)
    n_tok = B * H * W

    w1b = conv1_w.astype(jnp.bfloat16)
    w_all = jnp.concatenate(
        [conv2_w, jnp.zeros((7, CH), jnp.float32), cm_w[:, :, 0]],
        axis=0).astype(jnp.bfloat16)
    b_all = jnp.concatenate(
        [conv2_b, jnp.zeros((7,), jnp.float32), cm_b[:, 0]], axis=0)[:, None]

    col = lambda a: a[:, None]
    out, mask = pl.pallas_call(
        _make_kernel(n_tok),
        grid=(2, B, n_w),
        in_specs=[
            pl.BlockSpec((1, CIN, WT // 128, 128), lambda p, b, w: (b, 0, w, 0)),
            pl.BlockSpec((CH, CIN), lambda p, b, w: (0, 0)),
            pl.BlockSpec((CH, 1), lambda p, b, w: (0, 0)),
            pl.BlockSpec((CH, 1), lambda p, b, w: (0, 0)),
            pl.BlockSpec((CH, 1), lambda p, b, w: (0, 0)),
            pl.BlockSpec((2 * CLASSES + 8, CH), lambda p, b, w: (0, 0)),
            pl.BlockSpec((2 * CLASSES + 8, 1), lambda p, b, w: (0, 0)),
        ],
        out_specs=[
            pl.BlockSpec((1, 1, WT), lambda p, b, w: (b * n_w + w, 0, 0)),
            pl.BlockSpec((1, 1, WT), lambda p, b, w: (b * n_w + w, 0, 0)),
        ],
        out_shape=[
            jax.ShapeDtypeStruct((B * n_w, 1, WT), jnp.float32),
            jax.ShapeDtypeStruct((B * n_w, 1, WT), jnp.float32),
        ],
        scratch_shapes=[
            pltpu.VMEM((CH, 1), jnp.float32),
            pltpu.VMEM((CH, 1), jnp.float32),
            pltpu.VMEM((CH, 1), jnp.float32),
            pltpu.VMEM((CH, 1), jnp.float32),
        ],
        compiler_params=pltpu.CompilerParams(
            dimension_semantics=("arbitrary", "arbitrary", "arbitrary")),
    )(xv, w1b, col(conv1_b), col(bn_gamma), col(bn_beta), w_all, b_all)

    return out.reshape(B, 1, 1, W), mask.reshape(B, 1, 1, W)


# speedup vs baseline: 1.4722x; 1.1664x over previous
import jax
import jax.numpy as jnp
from jax.experimental import pallas as pl
from jax.experimental.pallas import tpu as pltpu

CLASSES = 128
CH = 128
EPS = 1e-5
WT = 4096


def _make_kernel(n_tok):
    inv_n = 1.0 / float(n_tok)

    def body(x_ref, w1_ref, b1_ref, g_ref, be_ref, wall_ref, ball_ref,
             out_ref, mask_ref, sy_ref, syy_ref, sc_ref, sh_ref):
        p = pl.program_id(0)
        first = jnp.logical_and(pl.program_id(1) == 0, pl.program_id(2) == 0)

        xt = x_ref[0].reshape(CH, WT).astype(jnp.bfloat16)
        y = jax.lax.dot_general(
            w1_ref[...], xt, (((1,), (0,)), ((), ())),
            preferred_element_type=jnp.float32) + b1_ref[...]

        @pl.when(p == 0)
        def _stats():
            @pl.when(first)
            def _():
                sy_ref[...] = jnp.zeros_like(sy_ref)
                syy_ref[...] = jnp.zeros_like(syy_ref)

            sy_ref[...] += jnp.sum(y, axis=1, keepdims=True)
            syy_ref[...] += jnp.sum(y * y, axis=1, keepdims=True)

        @pl.when(p == 1)
        def _fused():
            @pl.when(first)
            def _():
                mu = sy_ref[...] * inv_n
                var = syy_ref[...] * inv_n - mu * mu
                scale = g_ref[...] * jax.lax.rsqrt(var + EPS)
                sc_ref[...] = scale
                sh_ref[...] = be_ref[...] - mu * scale

            yn = y * sc_ref[...] + sh_ref[...]
            x_lat = jnp.where(yn >= 0, yn, 0.01 * yn).astype(jnp.bfloat16)
            z = jax.lax.dot_general(
                wall_ref[...], x_lat, (((1,), (0,)), ((), ())),
                preferred_element_type=jnp.float32) + ball_ref[...]
            logits = z[0:CLASSES, :]
            m = jnp.max(logits, axis=0, keepdims=True)
            row = jax.lax.broadcasted_iota(jnp.int32, logits.shape, 0)
            ind = jnp.min(jnp.where(logits == m, row, CLASSES),
                          axis=0, keepdims=True)
            allreg = z[CLASSES + 8:CLASSES + 8 + CLASSES, :]
            reg = jnp.sum(jnp.where(row == ind, allreg, 0.0),
                          axis=0, keepdims=True)
            out_ref[0] = (ind.astype(jnp.float32) + reg) * (1.0 / float(CLASSES))
            mz = z[CLASSES:CLASSES + 1, :]
            mask_ref[0] = jnp.where(mz >= 0, mz, 0.01 * mz)

    return body


def kernel(x, conv1_w, conv1_b, bn_gamma, bn_beta, conv2_w, conv2_b, cm_w, cm_b):
    B, CIN, H, W = x.shape
    n_w = W // WT
    xv = x.reshape(B, CIN, W // 128, 128)
    n_tok = B * H * W

    w1b = conv1_w.astype(jnp.bfloat16)
    w_all = jnp.concatenate(
        [conv2_w, jnp.zeros((7, CH), jnp.float32), cm_w[:, :, 0]],
        axis=0).astype(jnp.bfloat16)
    b_all = jnp.concatenate(
        [conv2_b, jnp.zeros((7,), jnp.float32), cm_b[:, 0]], axis=0)[:, None]

    col = lambda a: a[:, None]
    out, mask = pl.pallas_call(
        _make_kernel(n_tok),
        grid=(2, B, n_w),
        in_specs=[
            pl.BlockSpec((1, CIN, WT // 128, 128), lambda p, b, w: (b, 0, w, 0)),
            pl.BlockSpec((CH, CIN), lambda p, b, w: (0, 0)),
            pl.BlockSpec((CH, 1), lambda p, b, w: (0, 0)),
            pl.BlockSpec((CH, 1), lambda p, b, w: (0, 0)),
            pl.BlockSpec((CH, 1), lambda p, b, w: (0, 0)),
            pl.BlockSpec((2 * CLASSES + 8, CH), lambda p, b, w: (0, 0)),
            pl.BlockSpec((2 * CLASSES + 8, 1), lambda p, b, w: (0, 0)),
        ],
        out_specs=[
            pl.BlockSpec((1, 1, WT), lambda p, b, w: (b * n_w + w, 0, 0)),
            pl.BlockSpec((1, 1, WT), lambda p, b, w: (b * n_w + w, 0, 0)),
        ],
        out_shape=[
            jax.ShapeDtypeStruct((B * n_w, 1, WT), jnp.float32),
            jax.ShapeDtypeStruct((B * n_w, 1, WT), jnp.float32),
        ],
        scratch_shapes=[
            pltpu.VMEM((CH, 1), jnp.float32),
            pltpu.VMEM((CH, 1), jnp.float32),
            pltpu.VMEM((CH, 1), jnp.float32),
            pltpu.VMEM((CH, 1), jnp.float32),
        ],
        compiler_params=pltpu.CompilerParams(
            dimension_semantics=("arbitrary", "arbitrary", "arbitrary")),
    )(xv, w1b, col(conv1_b), col(bn_gamma), col(bn_beta), w_all, b_all)

    return out.reshape(B, 1, 1, W), mask.reshape(B, 1, 1, W)


# speedup vs baseline: 1.5914x; 1.0810x over previous
import jax
import jax.numpy as jnp
from jax.experimental import pallas as pl
from jax.experimental.pallas import tpu as pltpu

CLASSES = 128
CH = 128
EPS = 1e-5
WT = 8192


def _make_kernel(n_tok):
    inv_n = 1.0 / float(n_tok)

    def body(x_ref, w1_ref, b1_ref, g_ref, be_ref, wall_ref, ball_ref,
             out_ref, mask_ref, sy_ref, syy_ref, sc_ref, sh_ref):
        p = pl.program_id(0)
        first = jnp.logical_and(pl.program_id(1) == 0, pl.program_id(2) == 0)

        xt = x_ref[0].reshape(CH, WT).astype(jnp.bfloat16)
        y = jax.lax.dot_general(
            w1_ref[...], xt, (((1,), (0,)), ((), ())),
            preferred_element_type=jnp.float32) + b1_ref[...]

        @pl.when(p == 0)
        def _stats():
            @pl.when(first)
            def _():
                sy_ref[...] = jnp.zeros_like(sy_ref)
                syy_ref[...] = jnp.zeros_like(syy_ref)

            sy_ref[...] += jnp.sum(y, axis=1, keepdims=True)
            syy_ref[...] += jnp.sum(y * y, axis=1, keepdims=True)

        @pl.when(p == 1)
        def _fused():
            @pl.when(first)
            def _():
                mu = sy_ref[...] * inv_n
                var = syy_ref[...] * inv_n - mu * mu
                scale = g_ref[...] * jax.lax.rsqrt(var + EPS)
                sc_ref[...] = scale
                sh_ref[...] = be_ref[...] - mu * scale

            yn = y * sc_ref[...] + sh_ref[...]
            x_lat = jnp.where(yn >= 0, yn, 0.01 * yn).astype(jnp.bfloat16)
            z = jax.lax.dot_general(
                wall_ref[...], x_lat, (((1,), (0,)), ((), ())),
                preferred_element_type=jnp.float32) + ball_ref[...]
            logits = z[0:CLASSES, :]
            m = jnp.max(logits, axis=0, keepdims=True)
            row = jax.lax.broadcasted_iota(jnp.int32, logits.shape, 0)
            ind = jnp.min(jnp.where(logits == m, row, CLASSES),
                          axis=0, keepdims=True)
            allreg = z[CLASSES + 8:CLASSES + 8 + CLASSES, :]
            reg = jnp.sum(jnp.where(row == ind, allreg, 0.0),
                          axis=0, keepdims=True)
            out_ref[0] = (ind.astype(jnp.float32) + reg) * (1.0 / float(CLASSES))
            mz = z[CLASSES:CLASSES + 1, :]
            mask_ref[0] = jnp.where(mz >= 0, mz, 0.01 * mz)

    return body


def kernel(x, conv1_w, conv1_b, bn_gamma, bn_beta, conv2_w, conv2_b, cm_w, cm_b):
    B, CIN, H, W = x.shape
    n_w = W // WT
    xv = x.reshape(B, CIN, W // 128, 128)
    n_tok = B * H * W

    w1b = conv1_w.astype(jnp.bfloat16)
    w_all = jnp.concatenate(
        [conv2_w, jnp.zeros((7, CH), jnp.float32), cm_w[:, :, 0]],
        axis=0).astype(jnp.bfloat16)
    b_all = jnp.concatenate(
        [conv2_b, jnp.zeros((7,), jnp.float32), cm_b[:, 0]], axis=0)[:, None]

    col = lambda a: a[:, None]
    out, mask = pl.pallas_call(
        _make_kernel(n_tok),
        grid=(2, B, n_w),
        in_specs=[
            pl.BlockSpec((1, CIN, WT // 128, 128), lambda p, b, w: (b, 0, w, 0)),
            pl.BlockSpec((CH, CIN), lambda p, b, w: (0, 0)),
            pl.BlockSpec((CH, 1), lambda p, b, w: (0, 0)),
            pl.BlockSpec((CH, 1), lambda p, b, w: (0, 0)),
            pl.BlockSpec((CH, 1), lambda p, b, w: (0, 0)),
            pl.BlockSpec((2 * CLASSES + 8, CH), lambda p, b, w: (0, 0)),
            pl.BlockSpec((2 * CLASSES + 8, 1), lambda p, b, w: (0, 0)),
        ],
        out_specs=[
            pl.BlockSpec((1, 1, WT), lambda p, b, w: (b * n_w + w, 0, 0)),
            pl.BlockSpec((1, 1, WT), lambda p, b, w: (b * n_w + w, 0, 0)),
        ],
        out_shape=[
            jax.ShapeDtypeStruct((B * n_w, 1, WT), jnp.float32),
            jax.ShapeDtypeStruct((B * n_w, 1, WT), jnp.float32),
        ],
        scratch_shapes=[
            pltpu.VMEM((CH, 1), jnp.float32),
            pltpu.VMEM((CH, 1), jnp.float32),
            pltpu.VMEM((CH, 1), jnp.float32),
            pltpu.VMEM((CH, 1), jnp.float32),
        ],
        compiler_params=pltpu.CompilerParams(
            dimension_semantics=("arbitrary", "arbitrary", "arbitrary")),
    )(xv, w1b, col(conv1_b), col(bn_gamma), col(bn_beta), w_all, b_all)

    return out.reshape(B, 1, 1, W), mask.reshape(B, 1, 1, W)
